# baseline (speedup 1.0000x reference)
import dataclasses
import functools

import jax
import jax.numpy as jnp
from jax import lax
from jax.experimental import pallas as pl
from jax.experimental.pallas import tpu as pltpu
from jax.experimental.pallas import tpu_sc as plsc

N_NODES = 100000
N_EDGES = 6400000
N_GRAPHS = 64
N_PAD = 100096

NC, NS = 2, 16
NW = NC * NS
CHUNK = 128
SUP = 32
NSUP = 49
ROWS_PER_TILE = NSUP * SUP
R_TOTAL = NW * ROWS_PER_TILE
E_PAD = R_TOTAL * CHUNK
NZ = N_PAD // NS
ZROWS = 391

FW = 16
NBUF = 4

_GDN = lax.GatherDimensionNumbers(
    offset_dims=(), collapsed_slice_dims=(0,), start_index_map=(0,))


def _splat_lane(vec, zero16, j):
    idx = (zero16 + j).reshape(16, 1)
    return lax.gather(vec, idx, _GDN, (1,),
                      mode=lax.GatherScatterMode.PROMISE_IN_BOUNDS)

_mesh = plsc.VectorSubcoreMesh(
    core_axis_name="c", subcore_axis_name="s", num_cores=NC, num_subcores=NS
)

_sc_params = pltpu.CompilerParams()
if "needs_layout_passes" in pltpu.CompilerParams.__dataclass_fields__:
    _sc_params = dataclasses.replace(_sc_params, needs_layout_passes=False)
if "use_tc_tiling_on_sc" in pltpu.CompilerParams.__dataclass_fields__:
    _sc_params = dataclasses.replace(_sc_params, use_tc_tiling_on_sc=False)


def _spmm_body(y_hbm, src_hbm, dst_hbm, ew_hbm, out_hbm,
               sidx, didx, ewst, rowbuf, zbuf, acc, sem0, sem1, sem2,
               gsems, ssems):
    cid = lax.axis_index("c")
    sid = lax.axis_index("s")
    wid = cid * NS + sid

    @pl.loop(0, ZROWS)
    def _(i):
        zbuf[i] = jnp.zeros((FW,), jnp.float32)

    @pl.loop(0, NZ // ZROWS)
    def _(i):
        pltpu.sync_copy(zbuf, acc.at[pl.ds(sid * NZ + i * ZROWS, ZROWS)])

    plsc.subcore_barrier()

    base = wid * ROWS_PER_TILE

    zero16 = lax.iota(jnp.int32, 16) * 0

    def _scale_rows(rb, ewst, kk):
        @pl.loop(0, CHUNK // 16, unroll=4)
        def _(g):
            e_row = ewst[kk, pl.ds(g * 16, 16)]
            for j in range(16):
                e16 = _splat_lane(e_row, zero16, j)
                r = g * 16 + j
                rb[r] = rb[r] * e16

    def _gwait(b):
        pltpu.make_async_copy(
            y_hbm.at[sidx.at[0]], rowbuf.at[b], gsems.at[b]).wait()

    def _swait(b):
        pltpu.make_async_copy(
            rowbuf.at[b], acc.at[didx.at[0]], ssems.at[b]).wait()

    @pl.loop(0, NSUP)
    def _(s):
        r0 = base + s * SUP
        c1 = pltpu.async_copy(src_hbm.at[pl.ds(r0, SUP)], sidx, sem0)
        c2 = pltpu.async_copy(dst_hbm.at[pl.ds(r0, SUP)], didx, sem1)
        c3 = pltpu.async_copy(ew_hbm.at[pl.ds(r0, SUP)], ewst, sem2)
        c1.wait()
        c2.wait()
        c3.wait()

        pltpu.async_copy(y_hbm.at[sidx.at[0]], rowbuf.at[0], gsems.at[0])
        pltpu.async_copy(y_hbm.at[sidx.at[1]], rowbuf.at[1], gsems.at[1])

        @pl.loop(0, SUP, step=NBUF)
        def _(k):
            for b in range(NBUF):
                kk = k + b
                rb = rowbuf.at[b]
                _gwait(b)
                _scale_rows(rb, ewst, kk)
                pltpu.async_copy(rb, acc.at[didx.at[kk]], ssems.at[b],
                                 add=True)
                pb = (b + 2) % NBUF
                kk2 = kk + 2

                @pl.when(kk2 < SUP)
                def _():
                    @pl.when(kk2 >= NBUF)
                    def _():
                        _swait(pb)
                    pltpu.async_copy(y_hbm.at[sidx.at[kk2]], rowbuf.at[pb],
                                     gsems.at[pb])

        for b in range(NBUF):
            _swait(b)

    plsc.subcore_barrier()
    pltpu.sync_copy(acc.at[pl.ds(sid * NZ, NZ)],
                    out_hbm.at[cid, pl.ds(sid * NZ, NZ)])


@functools.partial(
    pl.kernel,
    out_type=jax.ShapeDtypeStruct((NC, N_PAD, FW), jnp.float32),
    mesh=_mesh,
    scratch_types=[
        pltpu.VMEM((SUP, CHUNK), jnp.int32),
        pltpu.VMEM((SUP, CHUNK), jnp.int32),
        pltpu.VMEM((SUP, CHUNK), jnp.float32),
        pltpu.VMEM((NBUF, CHUNK, FW), jnp.float32),
        pltpu.VMEM((ZROWS, FW), jnp.float32),
        pltpu.VMEM_SHARED((N_PAD, FW), jnp.float32),
        pltpu.SemaphoreType.DMA,
        pltpu.SemaphoreType.DMA,
        pltpu.SemaphoreType.DMA,
        pltpu.SemaphoreType.DMA((NBUF,)),
        pltpu.SemaphoreType.DMA((NBUF,)),
    ],
    compiler_params=_sc_params,
)
def _spmm(y_hbm, src_hbm, dst_hbm, ew_hbm, out_hbm,
          sidx, didx, ewst, rowbuf, zbuf, acc, sem0, sem1, sem2,
          gsems, ssems):
    _spmm_body(y_hbm, src_hbm, dst_hbm, ew_hbm, out_hbm,
               sidx, didx, ewst, rowbuf, zbuf, acc, sem0, sem1, sem2,
               gsems, ssems)


DROWS = 6272
DR_TILE = DROWS // NS


def _deg_body(dst_hbm, ew_hbm, idq_hbm, out_hbm,
              didx, ewst, degv, idqv, accd, sem0, sem1):
    cid = lax.axis_index("c")
    sid = lax.axis_index("s")
    wid = cid * NS + sid

    @pl.loop(0, DROWS)
    def _(i):
        degv[i] = jnp.zeros((FW,), jnp.float32)

    pltpu.sync_copy(idq_hbm, idqv)
    pltpu.sync_copy(degv.at[pl.ds(0, DR_TILE)],
                    accd.at[pl.ds(sid * DR_TILE, DR_TILE)])
    plsc.subcore_barrier()

    base = wid * ROWS_PER_TILE

    @pl.loop(0, NSUP)
    def _(s):
        r0 = base + s * SUP
        c1 = pltpu.async_copy(dst_hbm.at[pl.ds(r0, SUP)], didx, sem0)
        c2 = pltpu.async_copy(ew_hbm.at[pl.ds(r0, SUP)], ewst, sem1)
        c1.wait()
        c2.wait()

        @pl.loop(0, SUP)
        def _(kk):
            @pl.loop(0, CHUNK // 16)
            def _(g):
                d16 = didx[kk, pl.ds(g * 16, 16)]
                e16 = ewst[kk, pl.ds(g * 16, 16)]
                row16 = lax.shift_right_logical(d16, 4)
                col16 = lax.bitwise_and(d16, 15)
                plsc.addupdate_scatter(degv, [row16, col16], e16)

    @pl.loop(0, DROWS // CHUNK)
    def _(r):
        pltpu.sync_copy(degv.at[pl.ds(r * CHUNK, CHUNK)],
                        accd.at[idqv.at[r]], add=True)

    plsc.subcore_barrier()
    pltpu.sync_copy(accd.at[pl.ds(sid * DR_TILE, DR_TILE)],
                    out_hbm.at[cid, pl.ds(sid * DR_TILE, DR_TILE)])


@functools.partial(
    pl.kernel,
    out_type=jax.ShapeDtypeStruct((NC, DROWS, FW), jnp.float32),
    mesh=_mesh,
    scratch_types=[
        pltpu.VMEM((SUP, CHUNK), jnp.int32),
        pltpu.VMEM((SUP, CHUNK), jnp.float32),
        pltpu.VMEM((DROWS, FW), jnp.float32),
        pltpu.VMEM((DROWS // CHUNK, CHUNK), jnp.int32),
        pltpu.VMEM_SHARED((DROWS, FW), jnp.float32),
        pltpu.SemaphoreType.DMA,
        pltpu.SemaphoreType.DMA,
    ],
    compiler_params=_sc_params,
)
def _deg(dst_hbm, ew_hbm, idq_hbm, out_hbm,
         didx, ewst, degv, idqv, accd, sem0, sem1):
    _deg_body(dst_hbm, ew_hbm, idq_hbm, out_hbm,
              didx, ewst, degv, idqv, accd, sem0, sem1)


BLK = 6256
GRID = N_PAD // BLK


def _s1_body(d0_ref, d1_ref, x_ref, dis_ref, y1_ref):
    deg = d0_ref[...] + d1_ref[...] + 1.0
    dis = lax.rsqrt(deg)
    dis_ref[...] = jnp.broadcast_to(dis, (BLK, FW))
    xs = x_ref[...] * dis
    y1_ref[...] = jnp.concatenate(
        [xs, jnp.zeros((BLK, FW - 4), jnp.float32)], axis=1)


def _s2_body(z_ref, y1_ref, dis_ref, w1_ref, b1_ref, y2_ref):
    dis = dis_ref[...]
    pre = dis * (z_ref[0] + z_ref[1] + y1_ref[...])
    h1 = jnp.maximum(jnp.dot(pre[:, :4], w1_ref[...],
                             preferred_element_type=jnp.float32)
                     + b1_ref[...], 0.0)
    y2_ref[...] = dis * h1


def _s3_body(z_ref, y2_ref, dis_ref, w2_ref, b2_ref, w3_ref, y3_ref):
    dis = dis_ref[...]
    pre = dis * (z_ref[0] + z_ref[1] + y2_ref[...])
    h2 = jnp.maximum(jnp.dot(pre, w2_ref[...],
                             preferred_element_type=jnp.float32)
                     + b2_ref[...], 0.0)
    g = jnp.dot(h2, w3_ref[...], preferred_element_type=jnp.float32)
    y3_ref[...] = jnp.concatenate(
        [dis[:, :2] * g, jnp.zeros((BLK, FW - 2), jnp.float32)], axis=1)


def _s4_body(z_ref, y3_ref, dis_ref, b3_ref, batch_ref, sums_ref, out_ref):
    i = pl.program_id(0)

    @pl.when(i == 0)
    def _():
        sums_ref[...] = jnp.zeros((N_GRAPHS, 8), jnp.float32)

    h3 = (dis_ref[:, :2] * (z_ref[0, :, :2] + z_ref[1, :, :2]
                            + y3_ref[:, :2]) + b3_ref[...])
    vals = jnp.concatenate(
        [h3, jnp.ones((BLK, 1), jnp.float32), jnp.zeros((BLK, 5), jnp.float32)],
        axis=1)
    onehot = (batch_ref[...] ==
              lax.broadcasted_iota(jnp.int32, (1, N_GRAPHS), 1)
              ).astype(jnp.float32)
    contrib = lax.dot_general(onehot, vals, (((0,), (0,)), ((), ())),
                              preferred_element_type=jnp.float32)
    sums_ref[...] += contrib

    @pl.when(i == pl.num_programs(0) - 1)
    def _():
        s = sums_ref[...]
        out_ref[...] = s[:, :2] / jnp.maximum(s[:, 2:3], 1.0)


def _zspec():
    return pl.BlockSpec((NC, BLK, FW), lambda i: (0, i, 0))


def _nspec(w=FW):
    return pl.BlockSpec((BLK, w), lambda i: (i, 0))


def _full(shape):
    return pl.BlockSpec(shape, lambda i: tuple(0 for _ in shape))


def _s1(d0, d1, x):
    return pl.pallas_call(
        _s1_body,
        grid=(GRID,),
        in_specs=[_nspec(1), _nspec(1), _nspec(4)],
        out_specs=[_nspec(), _nspec()],
        out_shape=[jax.ShapeDtypeStruct((N_PAD, FW), jnp.float32),
                   jax.ShapeDtypeStruct((N_PAD, FW), jnp.float32)],
    )(d0, d1, x)


def _s2(z1, y1, dis16, W1, b1):
    return pl.pallas_call(
        _s2_body,
        grid=(GRID,),
        in_specs=[_zspec(), _nspec(), _nspec(), _full((4, 16)), _full((1, 16))],
        out_specs=_nspec(),
        out_shape=jax.ShapeDtypeStruct((N_PAD, FW), jnp.float32),
    )(z1, y1, dis16, W1, b1.reshape(1, 16))


def _s3(z2, y2, dis16, W2, b2, W3):
    return pl.pallas_call(
        _s3_body,
        grid=(GRID,),
        in_specs=[_zspec(), _nspec(), _nspec(), _full((16, 32)),
                  _full((1, 32)), _full((32, 2))],
        out_specs=_nspec(),
        out_shape=jax.ShapeDtypeStruct((N_PAD, FW), jnp.float32),
    )(z2, y2, dis16, W2, b2.reshape(1, 32), W3)


def _s4(z3, y3, dis16, b3, batch2):
    sums, out = pl.pallas_call(
        _s4_body,
        grid=(GRID,),
        in_specs=[_zspec(), _nspec(), _nspec(), _full((1, 2)),
                  pl.BlockSpec((BLK, 1), lambda i: (i, 0))],
        out_specs=[_full((N_GRAPHS, 8)), _full((N_GRAPHS, 2))],
        out_shape=[jax.ShapeDtypeStruct((N_GRAPHS, 8), jnp.float32),
                   jax.ShapeDtypeStruct((N_GRAPHS, 2), jnp.float32)],
    )(z3, y3, dis16, b3.reshape(1, 2), batch2)
    return out


def kernel(x, edge_index, edge_weight, batch, W1, b1, W2, b2, W3, b3):
    src = edge_index[0].astype(jnp.int32)
    dst = edge_index[1].astype(jnp.int32)
    ew = edge_weight.astype(jnp.float32)

    pad = E_PAD - N_EDGES
    srcp = jnp.concatenate([src, jnp.zeros((pad,), jnp.int32)]).reshape(
        R_TOTAL, CHUNK)
    dstp = jnp.concatenate([dst, jnp.zeros((pad,), jnp.int32)]).reshape(
        R_TOTAL, CHUNK)
    ewp = jnp.concatenate([ew, jnp.zeros((pad,), jnp.float32)]).reshape(
        R_TOTAL, CHUNK)

    xp = jnp.concatenate([x, jnp.zeros((N_PAD - N_NODES, 4), jnp.float32)])
    batch2 = jnp.concatenate(
        [batch.astype(jnp.int32),
         jnp.full((N_PAD - N_NODES,), N_GRAPHS, jnp.int32)]).reshape(N_PAD, 1)

    idq = jnp.arange(DROWS, dtype=jnp.int32).reshape(DROWS // CHUNK, CHUNK)
    zd = _deg(dstp, ewp, idq)
    d0 = zd[0].reshape(DROWS * FW)[:N_PAD].reshape(N_PAD, 1)
    d1 = zd[1].reshape(DROWS * FW)[:N_PAD].reshape(N_PAD, 1)
    dis16, y1 = _s1(d0, d1, xp)
    z1 = _spmm(y1, srcp, dstp, ewp)
    y2 = _s2(z1, y1, dis16, W1, b1)
    z2 = _spmm(y2, srcp, dstp, ewp)
    y3 = _s3(z2, y2, dis16, W2, b2, W3)
    z3 = _spmm(y3, srcp, dstp, ewp)
    return _s4(z3, y3, dis16, b3, batch2)


# speedup vs baseline: 1.0428x; 1.0428x over previous
import dataclasses
import functools

import jax
import jax.numpy as jnp
from jax import lax
from jax.experimental import pallas as pl
from jax.experimental.pallas import tpu as pltpu
from jax.experimental.pallas import tpu_sc as plsc

N_NODES = 100000
N_EDGES = 6400000
N_GRAPHS = 64
N_PAD = 100096

NC, NS = 2, 16
NW = NC * NS
CHUNK = 128
SUP = 28
NSUP = 56
ROWS_PER_TILE = NSUP * SUP
R_TOTAL = NW * ROWS_PER_TILE
E_PAD = R_TOTAL * CHUNK
NZ = N_PAD // NS
ZROWS = 391

FW = 16
NBUF = 4

_GDN = lax.GatherDimensionNumbers(
    offset_dims=(), collapsed_slice_dims=(0,), start_index_map=(0,))


def _splat_lane(vec, zero16, j):
    idx = (zero16 + j).reshape(16, 1)
    return lax.gather(vec, idx, _GDN, (1,),
                      mode=lax.GatherScatterMode.PROMISE_IN_BOUNDS)

_mesh = plsc.VectorSubcoreMesh(
    core_axis_name="c", subcore_axis_name="s", num_cores=NC, num_subcores=NS
)

_sc_params = pltpu.CompilerParams()
if "needs_layout_passes" in pltpu.CompilerParams.__dataclass_fields__:
    _sc_params = dataclasses.replace(_sc_params, needs_layout_passes=False)
if "use_tc_tiling_on_sc" in pltpu.CompilerParams.__dataclass_fields__:
    _sc_params = dataclasses.replace(_sc_params, use_tc_tiling_on_sc=False)


def _spmm_body(y_hbm, src_hbm, dst_hbm, ew_hbm, out_hbm,
               sidx, didx, ewst, rowbuf, acc, stsems, gsems, ssems):
    cid = lax.axis_index("c")
    sid = lax.axis_index("s")
    wid = cid * NS + sid

    rb0 = rowbuf.at[0]

    @pl.loop(0, CHUNK)
    def _(i):
        rb0[i] = jnp.zeros((FW,), jnp.float32)

    @pl.loop(0, NZ // CHUNK)
    def _(i):
        pltpu.sync_copy(rb0, acc.at[pl.ds(sid * NZ + i * CHUNK, CHUNK)])

    _rem = NZ - (NZ // CHUNK) * CHUNK
    if _rem:
        pltpu.sync_copy(
            rowbuf.at[0, pl.ds(0, _rem)],
            acc.at[pl.ds(sid * NZ + (NZ // CHUNK) * CHUNK, _rem)])

    plsc.subcore_barrier()

    base = wid * ROWS_PER_TILE

    zero16 = lax.iota(jnp.int32, 16) * 0

    def _scale_rows(rb, ewh, kk):
        @pl.loop(0, CHUNK // 16)
        def _(g):
            e_row = ewh[kk, pl.ds(g * 16, 16)]
            for j in range(16):
                e16 = _splat_lane(e_row, zero16, j)
                r = g * 16 + j
                rb[r] = rb[r] * e16

    def _gwait(b):
        pltpu.make_async_copy(
            y_hbm.at[sidx.at[0, 0]], rowbuf.at[b], gsems.at[b]).wait()

    def _swait(b):
        pltpu.make_async_copy(
            rowbuf.at[b], acc.at[didx.at[0, 0]], ssems.at[b]).wait()

    def _stage(s, h):
        r0 = base + s * SUP
        pltpu.async_copy(src_hbm.at[pl.ds(r0, SUP)], sidx.at[h],
                         stsems.at[h])
        pltpu.async_copy(dst_hbm.at[pl.ds(r0, SUP)], didx.at[h],
                         stsems.at[h])
        pltpu.async_copy(ew_hbm.at[pl.ds(r0, SUP)], ewst.at[h],
                         stsems.at[h])

    def _stage_wait(h):
        for _ in range(3):
            pltpu.make_async_copy(src_hbm.at[pl.ds(base, SUP)],
                                  sidx.at[h], stsems.at[h]).wait()

    _stage(0, 0)

    @pl.loop(0, NSUP, step=2)
    def _(s):
        for h in range(2):
            ss = s + h
            _stage_wait(h)

            @pl.when(ss + 1 < NSUP)
            def _():
                _stage(ss + 1, 1 - h)

            pltpu.async_copy(y_hbm.at[sidx.at[h, 0]], rowbuf.at[0],
                             gsems.at[0])
            pltpu.async_copy(y_hbm.at[sidx.at[h, 1]], rowbuf.at[1],
                             gsems.at[1])

            @pl.loop(0, SUP, step=NBUF)
            def _(k):
                for b in range(NBUF):
                    kk = k + b
                    rb = rowbuf.at[b]
                    _gwait(b)
                    _scale_rows(rb, ewst.at[h], kk)
                    pltpu.async_copy(rb, acc.at[didx.at[h, kk]],
                                     ssems.at[b], add=True)
                    pb = (b + 2) % NBUF
                    kk2 = kk + 2

                    @pl.when(kk2 < SUP)
                    def _():
                        @pl.when(kk2 >= NBUF)
                        def _():
                            _swait(pb)
                        pltpu.async_copy(y_hbm.at[sidx.at[h, kk2]],
                                         rowbuf.at[pb], gsems.at[pb])

            for b in range(NBUF):
                _swait(b)

    plsc.subcore_barrier()
    pltpu.sync_copy(acc.at[pl.ds(sid * NZ, NZ)],
                    out_hbm.at[cid, pl.ds(sid * NZ, NZ)])


@functools.partial(
    pl.kernel,
    out_type=jax.ShapeDtypeStruct((NC, N_PAD, FW), jnp.float32),
    mesh=_mesh,
    scratch_types=[
        pltpu.VMEM((2, SUP, CHUNK), jnp.int32),
        pltpu.VMEM((2, SUP, CHUNK), jnp.int32),
        pltpu.VMEM((2, SUP, CHUNK), jnp.float32),
        pltpu.VMEM((NBUF, CHUNK, FW), jnp.float32),
        pltpu.VMEM_SHARED((N_PAD, FW), jnp.float32),
        pltpu.SemaphoreType.DMA((2,)),
        pltpu.SemaphoreType.DMA((NBUF,)),
        pltpu.SemaphoreType.DMA((NBUF,)),
    ],
    compiler_params=_sc_params,
)
def _spmm(y_hbm, src_hbm, dst_hbm, ew_hbm, out_hbm,
          sidx, didx, ewst, rowbuf, acc, stsems, gsems, ssems):
    _spmm_body(y_hbm, src_hbm, dst_hbm, ew_hbm, out_hbm,
               sidx, didx, ewst, rowbuf, acc, stsems, gsems, ssems)


DROWS = 6272
DR_TILE = DROWS // NS


def _deg_body(dst_hbm, ew_hbm, idq_hbm, out_hbm,
              didx, ewst, degv, idqv, accd, sem0, sem1):
    cid = lax.axis_index("c")
    sid = lax.axis_index("s")
    wid = cid * NS + sid

    @pl.loop(0, DROWS)
    def _(i):
        degv[i] = jnp.zeros((FW,), jnp.float32)

    pltpu.sync_copy(idq_hbm, idqv)
    pltpu.sync_copy(degv.at[pl.ds(0, DR_TILE)],
                    accd.at[pl.ds(sid * DR_TILE, DR_TILE)])
    plsc.subcore_barrier()

    base = wid * ROWS_PER_TILE

    @pl.loop(0, NSUP)
    def _(s):
        r0 = base + s * SUP
        c1 = pltpu.async_copy(dst_hbm.at[pl.ds(r0, SUP)], didx, sem0)
        c2 = pltpu.async_copy(ew_hbm.at[pl.ds(r0, SUP)], ewst, sem1)
        c1.wait()
        c2.wait()

        @pl.loop(0, SUP)
        def _(kk):
            @pl.loop(0, CHUNK // 16)
            def _(g):
                d16 = didx[kk, pl.ds(g * 16, 16)]
                e16 = ewst[kk, pl.ds(g * 16, 16)]
                row16 = lax.shift_right_logical(d16, 4)
                col16 = lax.bitwise_and(d16, 15)
                plsc.addupdate_scatter(degv, [row16, col16], e16)

    @pl.loop(0, DROWS // CHUNK)
    def _(r):
        pltpu.sync_copy(degv.at[pl.ds(r * CHUNK, CHUNK)],
                        accd.at[idqv.at[r]], add=True)

    plsc.subcore_barrier()
    pltpu.sync_copy(accd.at[pl.ds(sid * DR_TILE, DR_TILE)],
                    out_hbm.at[cid, pl.ds(sid * DR_TILE, DR_TILE)])


@functools.partial(
    pl.kernel,
    out_type=jax.ShapeDtypeStruct((NC, DROWS, FW), jnp.float32),
    mesh=_mesh,
    scratch_types=[
        pltpu.VMEM((SUP, CHUNK), jnp.int32),
        pltpu.VMEM((SUP, CHUNK), jnp.float32),
        pltpu.VMEM((DROWS, FW), jnp.float32),
        pltpu.VMEM((DROWS // CHUNK, CHUNK), jnp.int32),
        pltpu.VMEM_SHARED((DROWS, FW), jnp.float32),
        pltpu.SemaphoreType.DMA,
        pltpu.SemaphoreType.DMA,
    ],
    compiler_params=_sc_params,
)
def _deg(dst_hbm, ew_hbm, idq_hbm, out_hbm,
         didx, ewst, degv, idqv, accd, sem0, sem1):
    _deg_body(dst_hbm, ew_hbm, idq_hbm, out_hbm,
              didx, ewst, degv, idqv, accd, sem0, sem1)


BLK = 6256
GRID = N_PAD // BLK


def _s1_body(d0_ref, d1_ref, x_ref, dis_ref, y1_ref):
    deg = d0_ref[...] + d1_ref[...] + 1.0
    dis = lax.rsqrt(deg)
    dis_ref[...] = jnp.broadcast_to(dis, (BLK, FW))
    xs = x_ref[...] * dis
    y1_ref[...] = jnp.concatenate(
        [xs, jnp.zeros((BLK, FW - 4), jnp.float32)], axis=1)


def _s2_body(z_ref, y1_ref, dis_ref, w1_ref, b1_ref, y2_ref):
    dis = dis_ref[...]
    pre = dis * (z_ref[0] + z_ref[1] + y1_ref[...])
    h1 = jnp.maximum(jnp.dot(pre[:, :4], w1_ref[...],
                             preferred_element_type=jnp.float32)
                     + b1_ref[...], 0.0)
    y2_ref[...] = dis * h1


def _s3_body(z_ref, y2_ref, dis_ref, w2_ref, b2_ref, w3_ref, y3_ref):
    dis = dis_ref[...]
    pre = dis * (z_ref[0] + z_ref[1] + y2_ref[...])
    h2 = jnp.maximum(jnp.dot(pre, w2_ref[...],
                             preferred_element_type=jnp.float32)
                     + b2_ref[...], 0.0)
    g = jnp.dot(h2, w3_ref[...], preferred_element_type=jnp.float32)
    y3_ref[...] = jnp.concatenate(
        [dis[:, :2] * g, jnp.zeros((BLK, FW - 2), jnp.float32)], axis=1)


def _s4_body(z_ref, y3_ref, dis_ref, b3_ref, batch_ref, sums_ref, out_ref):
    i = pl.program_id(0)

    @pl.when(i == 0)
    def _():
        sums_ref[...] = jnp.zeros((N_GRAPHS, 8), jnp.float32)

    h3 = (dis_ref[:, :2] * (z_ref[0, :, :2] + z_ref[1, :, :2]
                            + y3_ref[:, :2]) + b3_ref[...])
    vals = jnp.concatenate(
        [h3, jnp.ones((BLK, 1), jnp.float32), jnp.zeros((BLK, 5), jnp.float32)],
        axis=1)
    onehot = (batch_ref[...] ==
              lax.broadcasted_iota(jnp.int32, (1, N_GRAPHS), 1)
              ).astype(jnp.float32)
    contrib = lax.dot_general(onehot, vals, (((0,), (0,)), ((), ())),
                              preferred_element_type=jnp.float32)
    sums_ref[...] += contrib

    @pl.when(i == pl.num_programs(0) - 1)
    def _():
        s = sums_ref[...]
        out_ref[...] = s[:, :2] / jnp.maximum(s[:, 2:3], 1.0)


def _zspec():
    return pl.BlockSpec((NC, BLK, FW), lambda i: (0, i, 0))


def _nspec(w=FW):
    return pl.BlockSpec((BLK, w), lambda i: (i, 0))


def _full(shape):
    return pl.BlockSpec(shape, lambda i: tuple(0 for _ in shape))


def _s1(d0, d1, x):
    return pl.pallas_call(
        _s1_body,
        grid=(GRID,),
        in_specs=[_nspec(1), _nspec(1), _nspec(4)],
        out_specs=[_nspec(), _nspec()],
        out_shape=[jax.ShapeDtypeStruct((N_PAD, FW), jnp.float32),
                   jax.ShapeDtypeStruct((N_PAD, FW), jnp.float32)],
    )(d0, d1, x)


def _s2(z1, y1, dis16, W1, b1):
    return pl.pallas_call(
        _s2_body,
        grid=(GRID,),
        in_specs=[_zspec(), _nspec(), _nspec(), _full((4, 16)), _full((1, 16))],
        out_specs=_nspec(),
        out_shape=jax.ShapeDtypeStruct((N_PAD, FW), jnp.float32),
    )(z1, y1, dis16, W1, b1.reshape(1, 16))


def _s3(z2, y2, dis16, W2, b2, W3):
    return pl.pallas_call(
        _s3_body,
        grid=(GRID,),
        in_specs=[_zspec(), _nspec(), _nspec(), _full((16, 32)),
                  _full((1, 32)), _full((32, 2))],
        out_specs=_nspec(),
        out_shape=jax.ShapeDtypeStruct((N_PAD, FW), jnp.float32),
    )(z2, y2, dis16, W2, b2.reshape(1, 32), W3)


def _s4(z3, y3, dis16, b3, batch2):
    sums, out = pl.pallas_call(
        _s4_body,
        grid=(GRID,),
        in_specs=[_zspec(), _nspec(), _nspec(), _full((1, 2)),
                  pl.BlockSpec((BLK, 1), lambda i: (i, 0))],
        out_specs=[_full((N_GRAPHS, 8)), _full((N_GRAPHS, 2))],
        out_shape=[jax.ShapeDtypeStruct((N_GRAPHS, 8), jnp.float32),
                   jax.ShapeDtypeStruct((N_GRAPHS, 2), jnp.float32)],
    )(z3, y3, dis16, b3.reshape(1, 2), batch2)
    return out


def kernel(x, edge_index, edge_weight, batch, W1, b1, W2, b2, W3, b3):
    src = edge_index[0].astype(jnp.int32)
    dst = edge_index[1].astype(jnp.int32)
    ew = edge_weight.astype(jnp.float32)

    pad = E_PAD - N_EDGES
    srcp = jnp.concatenate([src, jnp.zeros((pad,), jnp.int32)]).reshape(
        R_TOTAL, CHUNK)
    dstp = jnp.concatenate([dst, jnp.zeros((pad,), jnp.int32)]).reshape(
        R_TOTAL, CHUNK)
    ewp = jnp.concatenate([ew, jnp.zeros((pad,), jnp.float32)]).reshape(
        R_TOTAL, CHUNK)

    xp = jnp.concatenate([x, jnp.zeros((N_PAD - N_NODES, 4), jnp.float32)])
    batch2 = jnp.concatenate(
        [batch.astype(jnp.int32),
         jnp.full((N_PAD - N_NODES,), N_GRAPHS, jnp.int32)]).reshape(N_PAD, 1)

    idq = jnp.arange(DROWS, dtype=jnp.int32).reshape(DROWS // CHUNK, CHUNK)
    zd = _deg(dstp, ewp, idq)
    d0 = zd[0].reshape(DROWS * FW)[:N_PAD].reshape(N_PAD, 1)
    d1 = zd[1].reshape(DROWS * FW)[:N_PAD].reshape(N_PAD, 1)
    dis16, y1 = _s1(d0, d1, xp)
    z1 = _spmm(y1, srcp, dstp, ewp)
    y2 = _s2(z1, y1, dis16, W1, b1)
    z2 = _spmm(y2, srcp, dstp, ewp)
    y3 = _s3(z2, y2, dis16, W2, b2, W3)
    z3 = _spmm(y3, srcp, dstp, ewp)
    return _s4(z3, y3, dis16, b3, batch2)
